# Initial kernel scaffold; baseline (speedup 1.0000x reference)
import jax
import jax.numpy as jnp
from jax.experimental import pallas as pl


def kernel(inputs, word_embeddings, P):
    raise NotImplementedError("write your pallas kernel here")


# speedup vs baseline: 1.7970x; 1.7970x over previous
import jax
import jax.numpy as jnp
from jax.experimental import pallas as pl


def _add_body(we_ref, p_ref, out_ref):
    out_ref[...] = we_ref[...] + p_ref[...][None, :, :]


def kernel(inputs, word_embeddings, P):
    del inputs
    B, S, D = word_embeddings.shape
    if P.shape[0] != S:
        P = P[:S]
    SBLK = 512
    grid = (S // SBLK,)
    return pl.pallas_call(
        _add_body,
        grid=grid,
        in_specs=[
            pl.BlockSpec((B, SBLK, D), lambda i: (0, i, 0)),
            pl.BlockSpec((SBLK, D), lambda i: (i, 0)),
        ],
        out_specs=pl.BlockSpec((B, SBLK, D), lambda i: (0, i, 0)),
        out_shape=jax.ShapeDtypeStruct((B, S, D), word_embeddings.dtype),
    )(word_embeddings, P)
